# baseline (speedup 1.0000x reference)
import jax
import jax.numpy as jnp
from jax import lax
from jax.experimental import pallas as pl
from jax.experimental.pallas import tpu as pltpu
from jax.experimental.pallas import tpu_sc as plsc

NUM_CORES = 2
NUM_SUBCORES = 16
LANES = 16
NUM_WORKERS = NUM_CORES * NUM_SUBCORES
DIM = 32
BLK = 128
NBUF = 8
B_PER_W = 512


def _sc_kernel(uidx_hbm, iidx_hbm, utab_hbm, itab_hbm, params_hbm, out_hbm,
               idx_u, idx_i, u_st0, u_st1, u_st2, u_st3,
               u_st4, u_st5, u_st6, u_st7,
               i_st0, i_st1, i_st2, i_st3,
               i_st4, i_st5, i_st6, i_st7, params_v, out_v,
               sem_u0, sem_u1, sem_u2, sem_u3,
               sem_u4, sem_u5, sem_u6, sem_u7,
               sem_i0, sem_i1, sem_i2, sem_i3,
               sem_i4, sem_i5, sem_i6, sem_i7):
    wid = lax.axis_index("s") * NUM_CORES + lax.axis_index("c")
    base = wid * B_PER_W

    pltpu.sync_copy(params_hbm, params_v)
    pltpu.sync_copy(uidx_hbm.at[pl.ds(base, B_PER_W)],
                    idx_u.at[pl.ds(0, B_PER_W)])
    pltpu.sync_copy(iidx_hbm.at[pl.ds(base, B_PER_W)],
                    idx_i.at[pl.ds(0, B_PER_W)])

    u_bufs = (u_st0, u_st1, u_st2, u_st3, u_st4, u_st5, u_st6, u_st7)
    i_bufs = (i_st0, i_st1, i_st2, i_st3, i_st4, i_st5, i_st6, i_st7)
    u_sems = (sem_u0, sem_u1, sem_u2, sem_u3,
              sem_u4, sem_u5, sem_u6, sem_u7)
    i_sems = (sem_i0, sem_i1, sem_i2, sem_i3,
              sem_i4, sem_i5, sem_i6, sem_i7)

    def issue(b, buf_id):
        iv_u = idx_u[pl.ds(b, LANES)]
        iv_i = idx_i[pl.ds(b, LANES)]
        bu = pl.multiple_of((iv_u[0] // BLK) * BLK, BLK)
        bi = pl.multiple_of((iv_i[0] // BLK) * BLK, BLK)
        pltpu.async_copy(utab_hbm.at[:, pl.ds(bu, BLK)],
                         u_bufs[buf_id], u_sems[buf_id])
        pltpu.async_copy(itab_hbm.at[:, pl.ds(bi, BLK)],
                         i_bufs[buf_id], i_sems[buf_id])

    def wait(buf_id):
        pltpu.make_async_copy(utab_hbm.at[:, pl.ds(0, BLK)],
                              u_bufs[buf_id], u_sems[buf_id]).wait()
        pltpu.make_async_copy(itab_hbm.at[:, pl.ds(0, BLK)],
                              i_bufs[buf_id], i_sems[buf_id]).wait()

    w_lo = params_v[pl.ds(0, LANES)]
    w_hi = params_v[pl.ds(LANES, LANES)]
    bias = params_v[pl.ds(DIM, LANES)][0]
    iota16 = lax.iota(jnp.int32, LANES)
    rows_lo = iota16
    rows_hi = iota16 + LANES

    def compute(b, buf_id, vals):
        ub = u_bufs[buf_id]
        ib = i_bufs[buf_id]
        iv_u = idx_u[pl.ds(b, LANES)]
        iv_i = idx_i[pl.ds(b, LANES)]
        cu = jnp.full((LANES,), iv_u[0] % BLK, jnp.int32)
        ci = jnp.full((LANES,), iv_i[0] % BLK, jnp.int32)
        u0 = plsc.load_gather(ub, [rows_lo, cu])
        u1 = plsc.load_gather(ub, [rows_hi, cu])
        i0 = plsc.load_gather(ib, [rows_lo, ci])
        i1 = plsc.load_gather(ib, [rows_hi, ci])
        s = jnp.sum(u0 * i0 * w_lo + u1 * i1 * w_hi)
        return jnp.where(iota16 == b % LANES, s, vals)

    for j in range(NBUF - 1):
        issue(j, j)
    zeros = jnp.zeros((LANES,), jnp.float32)

    def body(t, vals):
        b0 = NBUF * t
        vals = jnp.where((t % 2) == 0, zeros, vals)
        for j in range(NBUF):
            b = b0 + j

            @pl.when(b + NBUF - 1 < B_PER_W)
            def _():
                issue(b + NBUF - 1, (j + NBUF - 1) % NBUF)

            wait(j)
            vals = compute(b, j, vals)

        @pl.when((t % 2) == 1)
        def _():
            out_v[pl.ds((t // 2) * LANES, LANES)] = vals + bias

        return vals

    lax.fori_loop(0, B_PER_W // NBUF, body, zeros)

    pltpu.sync_copy(out_v, out_hbm.at[pl.ds(base, B_PER_W)])


def kernel(user_indices, item_indices, user_table, item_table, fc_w, fc_b):
    batch = user_indices.shape[0]
    params = jnp.concatenate(
        [fc_w.reshape(DIM), fc_b.reshape(1),
         jnp.zeros((15,), jnp.float32)]).astype(jnp.float32)

    mesh = plsc.VectorSubcoreMesh(core_axis_name="c", subcore_axis_name="s")
    stage = pltpu.VMEM((DIM, BLK), jnp.float32)
    sem = pltpu.SemaphoreType.DMA
    run = pl.kernel(
        _sc_kernel,
        out_type=jax.ShapeDtypeStruct((batch,), jnp.float32),
        mesh=mesh,
        compiler_params=pltpu.CompilerParams(
            needs_layout_passes=False, use_tc_tiling_on_sc=True),
        scratch_types=[
            pltpu.VMEM((B_PER_W + LANES,), jnp.int32),
            pltpu.VMEM((B_PER_W + LANES,), jnp.int32),
            stage, stage, stage, stage, stage, stage, stage, stage,
            stage, stage, stage, stage, stage, stage, stage, stage,
            pltpu.VMEM((DIM + 16,), jnp.float32),
            pltpu.VMEM((B_PER_W,), jnp.float32),
            sem, sem, sem, sem, sem, sem, sem, sem,
            sem, sem, sem, sem, sem, sem, sem, sem,
        ],
    )
    return run(user_indices.astype(jnp.int32), item_indices.astype(jnp.int32),
               user_table.T, item_table.T, params)


# speedup vs baseline: 1.0562x; 1.0562x over previous
import jax
import jax.numpy as jnp
from jax import lax
from jax.experimental import pallas as pl
from jax.experimental.pallas import tpu as pltpu
from jax.experimental.pallas import tpu_sc as plsc

NUM_CORES = 2
NUM_SUBCORES = 16
LANES = 16
NUM_WORKERS = NUM_CORES * NUM_SUBCORES
DIM = 32
BLK = 128
CH = 2
NBUF = 4
B_PER_W = 512
N_CHUNKS = B_PER_W // CH


def _sc_kernel(uidx_hbm, iidx_hbm, utab_hbm, itab_hbm, params_hbm, out_hbm,
               idx_u, idx_i, u_st0, u_st1, u_st2, u_st3,
               i_st0, i_st1, i_st2, i_st3, params_v, out_v,
               sem_u0, sem_u1, sem_u2, sem_u3,
               sem_i0, sem_i1, sem_i2, sem_i3):
    wid = lax.axis_index("s") * NUM_CORES + lax.axis_index("c")
    base = wid * B_PER_W

    pltpu.sync_copy(params_hbm, params_v)
    pltpu.sync_copy(uidx_hbm.at[pl.ds(base, B_PER_W)],
                    idx_u.at[pl.ds(0, B_PER_W)])
    pltpu.sync_copy(iidx_hbm.at[pl.ds(base, B_PER_W)],
                    idx_i.at[pl.ds(0, B_PER_W)])

    u_bufs = (u_st0, u_st1, u_st2, u_st3)
    i_bufs = (i_st0, i_st1, i_st2, i_st3)
    u_sems = (sem_u0, sem_u1, sem_u2, sem_u3)
    i_sems = (sem_i0, sem_i1, sem_i2, sem_i3)

    def issue_chunk(c, buf_id):
        ub = u_bufs[buf_id]
        ib = i_bufs[buf_id]
        iv_u = idx_u[pl.ds(c * CH, LANES)]
        iv_i = idx_i[pl.ds(c * CH, LANES)]
        for k in range(CH):
            bu = pl.multiple_of((iv_u[k] // BLK) * BLK, BLK)
            bi = pl.multiple_of((iv_i[k] // BLK) * BLK, BLK)
            pltpu.async_copy(utab_hbm.at[:, pl.ds(bu, BLK)],
                             ub.at[:, pl.ds(k * BLK, BLK)], u_sems[buf_id])
            pltpu.async_copy(itab_hbm.at[:, pl.ds(bi, BLK)],
                             ib.at[:, pl.ds(k * BLK, BLK)], i_sems[buf_id])

    def wait_chunk(buf_id):
        pltpu.make_async_copy(utab_hbm.at[:, pl.ds(0, CH * BLK)],
                              u_bufs[buf_id], u_sems[buf_id]).wait()
        pltpu.make_async_copy(itab_hbm.at[:, pl.ds(0, CH * BLK)],
                              i_bufs[buf_id], i_sems[buf_id]).wait()

    w_lo = params_v[pl.ds(0, LANES)]
    w_hi = params_v[pl.ds(LANES, LANES)]
    bias = params_v[pl.ds(DIM, LANES)][0]
    iota16 = lax.iota(jnp.int32, LANES)
    rows_lo = iota16
    rows_hi = iota16 + LANES

    def compute_chunk(c, buf_id, j, vals):
        ub = u_bufs[buf_id]
        ib = i_bufs[buf_id]
        iv_u = idx_u[pl.ds(c * CH, LANES)]
        iv_i = idx_i[pl.ds(c * CH, LANES)]
        for k in range(CH):
            cu = jnp.full((LANES,), (iv_u[k] % BLK) + k * BLK, jnp.int32)
            ci = jnp.full((LANES,), (iv_i[k] % BLK) + k * BLK, jnp.int32)
            u0 = plsc.load_gather(ub, [rows_lo, cu])
            u1 = plsc.load_gather(ub, [rows_hi, cu])
            i0 = plsc.load_gather(ib, [rows_lo, ci])
            i1 = plsc.load_gather(ib, [rows_hi, ci])
            s = jnp.sum(u0 * i0 * w_lo + u1 * i1 * w_hi)
            vals = jnp.where(iota16 == (c * CH + k) % LANES, s, vals)
        return vals

    issue_chunk(0, 0)
    issue_chunk(1, 1)
    issue_chunk(2, 2)
    zeros = jnp.zeros((LANES,), jnp.float32)

    def body(t, vals):
        c0 = 4 * t
        vals = jnp.where((t % 2) == 0, zeros, vals)
        for j in range(4):
            c = c0 + j

            @pl.when(c + 3 < N_CHUNKS)
            def _():
                issue_chunk(c + 3, (j + 3) % 4)

            wait_chunk(j)
            vals = compute_chunk(c, j, j, vals)

        @pl.when((t % 2) == 1)
        def _():
            out_v[pl.ds((t // 2) * LANES, LANES)] = vals + bias

        return vals

    lax.fori_loop(0, N_CHUNKS // 4, body, zeros)

    pltpu.sync_copy(out_v, out_hbm.at[pl.ds(base, B_PER_W)])


def kernel(user_indices, item_indices, user_table, item_table, fc_w, fc_b):
    batch = user_indices.shape[0]
    params = jnp.concatenate(
        [fc_w.reshape(DIM), fc_b.reshape(1),
         jnp.zeros((15,), jnp.float32)]).astype(jnp.float32)

    mesh = plsc.VectorSubcoreMesh(core_axis_name="c", subcore_axis_name="s")
    stage = pltpu.VMEM((DIM, CH * BLK), jnp.float32)
    run = pl.kernel(
        _sc_kernel,
        out_type=jax.ShapeDtypeStruct((batch,), jnp.float32),
        mesh=mesh,
        compiler_params=pltpu.CompilerParams(
            needs_layout_passes=False, use_tc_tiling_on_sc=True),
        scratch_types=[
            pltpu.VMEM((B_PER_W + LANES,), jnp.int32),
            pltpu.VMEM((B_PER_W + LANES,), jnp.int32),
            stage, stage, stage, stage,
            stage, stage, stage, stage,
            pltpu.VMEM((DIM + 16,), jnp.float32),
            pltpu.VMEM((B_PER_W,), jnp.float32),
            pltpu.SemaphoreType.DMA, pltpu.SemaphoreType.DMA,
            pltpu.SemaphoreType.DMA, pltpu.SemaphoreType.DMA,
            pltpu.SemaphoreType.DMA, pltpu.SemaphoreType.DMA,
            pltpu.SemaphoreType.DMA, pltpu.SemaphoreType.DMA,
        ],
    )
    return run(user_indices.astype(jnp.int32), item_indices.astype(jnp.int32),
               user_table.T, item_table.T, params)
